# Initial kernel scaffold; baseline (speedup 1.0000x reference)
import jax
import jax.numpy as jnp
from jax.experimental import pallas as pl


def kernel(char_ids, word_ids, char_table, word_table):
    raise NotImplementedError("write your pallas kernel here")


# speedup vs baseline: 15.0351x; 15.0351x over previous
import functools

import jax
import jax.numpy as jnp
from jax import lax
from jax.experimental import pallas as pl
from jax.experimental.pallas import tpu as pltpu
from jax.experimental.pallas import tpu_sc as plsc

_NCHARS = 256
_CE = 8
_WE = 16
_WLEN = 20
_OUT_D = _WE + _WLEN * _CE
_NTILES = 32
_K = 128


def _build_kernel(n_pairs: int):
    assert n_pairs % (_NTILES * _K) == 0
    p_per_tile = n_pairs // _NTILES
    n_blocks = p_per_tile // _K
    mesh = plsc.VectorSubcoreMesh(core_axis_name="c", subcore_axis_name="s")

    @functools.partial(
        pl.kernel,
        out_type=jax.ShapeDtypeStruct((n_pairs * _OUT_D,), jnp.float32),
        mesh=mesh,
        scratch_types=[
            pltpu.VMEM(((_NCHARS + 1) * _CE,), jnp.float32),
            pltpu.VMEM((_K * _WLEN,), jnp.int32),
            pltpu.VMEM((_K,), jnp.int32),
            pltpu.VMEM((_K, _WE), jnp.float32),
            pltpu.VMEM((_K * _OUT_D,), jnp.float32),
            pltpu.SemaphoreType.DMA,
        ],
        compiler_params=pltpu.CompilerParams(
            needs_layout_passes=False, use_tc_tiling_on_sc=False),
    )
    def _k(cid_hbm, wid_hbm, ctab_hbm, wtab_hbm, out_hbm,
           ctab_v, cid_v, wid_v, wrows_v, outb_v, sem):
        tid = lax.axis_index("s") * 2 + lax.axis_index("c")
        tbase = tid * p_per_tile

        iota = lax.iota(jnp.int32, 16)
        lo8 = iota & 7
        hi8 = iota >> 3

        pltpu.sync_copy(ctab_hbm, ctab_v)
        head = ctab_v[pl.ds(0, 16)]
        ctab_v[pl.ds(0, 16)] = jnp.where(iota < _CE, 0.0, head)

        zeros = jnp.zeros((16,), jnp.float32)

        def block(b, carry):
            pbase = tbase + b * _K
            pltpu.sync_copy(cid_hbm.at[pl.ds(pbase * _WLEN, _K * _WLEN)], cid_v)
            pltpu.sync_copy(wid_hbm.at[pl.ds(pbase, _K)], wid_v)
            pltpu.async_copy(wtab_hbm.at[wid_v], wrows_v, sem).wait()

            def pair(p, c):
                ob = p * _OUT_D
                cb = p * _WLEN
                outb_v[pl.ds(ob, 16)] = wrows_v[p]
                for j in range(_WLEN // 2):
                    cidj = plsc.load_gather(cid_v, [cb + 2 * j + hi8])
                    vals = plsc.load_gather(ctab_v, [cidj * _CE + lo8])
                    outb_v[pl.ds(ob + _WE + 16 * j, 16)] = vals
                return c
            lax.fori_loop(0, _K, pair, 0)

            def zgrp(g, c):
                widv = wid_v[pl.ds(g * 16, 16)]
                m = widv == 0
                obase = (g * 16 + iota) * _OUT_D
                for col in range(_WE):
                    plsc.store_scatter(outb_v, [obase + col], zeros, mask=m)
                return c
            lax.fori_loop(0, _K // 16, zgrp, 0)

            pltpu.sync_copy(outb_v, out_hbm.at[pl.ds(pbase * _OUT_D, _K * _OUT_D)])
            return carry

        lax.fori_loop(0, n_blocks, block, 0)

    return _k


@jax.jit
def kernel(char_ids, word_ids, char_table, word_table):
    b, s, w = char_ids.shape
    n_pairs = b * s
    k = _build_kernel(n_pairs)
    out = k(char_ids.reshape(-1), word_ids.reshape(-1),
            char_table.reshape(-1), word_table)
    return out.reshape(b, s, _OUT_D)


# speedup vs baseline: 16.3055x; 1.0845x over previous
import functools

import jax
import jax.numpy as jnp
from jax import lax
from jax.experimental import pallas as pl
from jax.experimental.pallas import tpu as pltpu
from jax.experimental.pallas import tpu_sc as plsc

_NCHARS = 256
_CE = 8
_WE = 16
_WLEN = 20
_OUT_D = _WE + _WLEN * _CE
_NTILES = 32
_K = 128


def _build_kernel(n_pairs: int):
    assert n_pairs % (_NTILES * 2 * _K) == 0
    p_per_tile = n_pairs // _NTILES
    n_blocks = p_per_tile // _K
    mesh = plsc.VectorSubcoreMesh(core_axis_name="c", subcore_axis_name="s")

    @functools.partial(
        pl.kernel,
        out_type=jax.ShapeDtypeStruct((n_pairs * _OUT_D,), jnp.float32),
        mesh=mesh,
        scratch_types=[
            pltpu.VMEM(((_NCHARS + 1) * _CE,), jnp.float32),
            pltpu.VMEM((_K * _WLEN,), jnp.int32),
            pltpu.VMEM((_K * _WLEN,), jnp.int32),
            pltpu.VMEM((_K,), jnp.int32),
            pltpu.VMEM((_K,), jnp.int32),
            pltpu.VMEM((_K, _WE), jnp.float32),
            pltpu.VMEM((_K, _WE), jnp.float32),
            pltpu.VMEM((_K * _OUT_D,), jnp.float32),
            pltpu.VMEM((_K * _OUT_D,), jnp.float32),
            pltpu.SemaphoreType.DMA,
            pltpu.SemaphoreType.DMA,
            pltpu.SemaphoreType.DMA,
            pltpu.SemaphoreType.DMA,
            pltpu.SemaphoreType.DMA,
            pltpu.SemaphoreType.DMA,
            pltpu.SemaphoreType.DMA,
        ],
        compiler_params=pltpu.CompilerParams(
            needs_layout_passes=False, use_tc_tiling_on_sc=False),
    )
    def _k(cid_hbm, wid_hbm, ctab_hbm, wtab_hbm, out_hbm,
           ctab_v, cid0_v, cid1_v, wid0_v, wid1_v, wrows0_v, wrows1_v,
           outb0_v, outb1_v, tsem, isem0, isem1, gsem0, gsem1, osem0, osem1):
        cid_b = (cid0_v, cid1_v)
        wid_b = (wid0_v, wid1_v)
        wrows_b = (wrows0_v, wrows1_v)
        outb_b = (outb0_v, outb1_v)
        isem_b = (isem0, isem1)
        gsem_b = (gsem0, gsem1)
        osem_b = (osem0, osem1)
        tid = lax.axis_index("s") * 2 + lax.axis_index("c")
        tbase = tid * p_per_tile

        iota = lax.iota(jnp.int32, 16)
        lo8 = iota & 7
        hi8 = iota >> 3
        zeros = jnp.zeros((16,), jnp.float32)

        pltpu.async_copy(ctab_hbm, ctab_v, tsem).wait()
        head = ctab_v[pl.ds(0, 16)]
        ctab_v[pl.ds(0, 16)] = jnp.where(iota < _CE, 0.0, head)

        def start_ids(b, h):
            pbase = tbase + b * _K
            pltpu.async_copy(
                cid_hbm.at[pl.ds(pbase * _WLEN, _K * _WLEN)], cid_b[h],
                isem_b[h])
            pltpu.async_copy(
                wid_hbm.at[pl.ds(pbase, _K)], wid_b[h], isem_b[h])

        def wait_ids(h):
            pltpu.make_async_copy(
                cid_hbm.at[pl.ds(0, _K * _WLEN)], cid_b[h], isem_b[h]).wait()
            pltpu.make_async_copy(
                wid_hbm.at[pl.ds(0, _K)], wid_b[h], isem_b[h]).wait()

        def start_gather(h):
            pltpu.async_copy(wtab_hbm.at[wid_b[h]], wrows_b[h], gsem_b[h])

        def wait_gather(h):
            pltpu.make_async_copy(wtab_hbm.at[wid_b[h]], wrows_b[h],
                                  gsem_b[h]).wait()

        def start_out(b, h):
            pbase = tbase + b * _K
            pltpu.async_copy(
                outb_b[h],
                out_hbm.at[pl.ds(pbase * _OUT_D, _K * _OUT_D)], osem_b[h])

        def wait_out(b, h):
            pbase = tbase + b * _K
            pltpu.make_async_copy(
                outb_b[h],
                out_hbm.at[pl.ds(pbase * _OUT_D, _K * _OUT_D)],
                osem_b[h]).wait()

        start_ids(0, 0)
        wait_ids(0)
        start_gather(0)
        start_ids(1, 1)

        def block2(b2, carry):
            for h in (0, 1):
                b = b2 * 2 + h
                wait_gather(h)
                @pl.when(b >= 2)
                def _():
                    wait_out(b - 2, h)

                cid_h = cid_b[h]
                wid_h = wid_b[h]
                wrows_h = wrows_b[h]
                outb_h = outb_b[h]

                def pair(p, c):
                    ob = p * _OUT_D
                    cb = p * _WLEN
                    outb_h[pl.ds(ob, 16)] = wrows_h[p]
                    for j in range(_WLEN // 2):
                        cidj = plsc.load_gather(cid_h, [cb + 2 * j + hi8])
                        vals = plsc.load_gather(ctab_h_ref, [cidj * _CE + lo8])
                        outb_h[pl.ds(ob + _WE + 16 * j, 16)] = vals
                    return c
                ctab_h_ref = ctab_v
                lax.fori_loop(0, _K, pair, 0, unroll=4)

                def zgrp(g, c):
                    widv = wid_h[pl.ds(g * 16, 16)]
                    m = widv == 0
                    obase = (g * 16 + iota) * _OUT_D
                    for col in range(_WE):
                        plsc.store_scatter(outb_h, [obase + col], zeros,
                                           mask=m)
                    return c
                lax.fori_loop(0, _K // 16, zgrp, 0, unroll=2)

                start_out(b, h)

                @pl.when(b + 1 < n_blocks)
                def _():
                    wait_ids(1 - h)
                    start_gather(1 - h)

                @pl.when(b + 2 < n_blocks)
                def _():
                    start_ids(b + 2, h)
            return carry

        lax.fori_loop(0, n_blocks // 2, block2, 0)

        wait_out(n_blocks - 2, 0)
        wait_out(n_blocks - 1, 1)

    return _k


@jax.jit
def kernel(char_ids, word_ids, char_table, word_table):
    b, s, w = char_ids.shape
    n_pairs = b * s
    k = _build_kernel(n_pairs)
    out = k(char_ids.reshape(-1), word_ids.reshape(-1),
            char_table.reshape(-1), word_table)
    return out.reshape(b, s, _OUT_D)


# speedup vs baseline: 19.2407x; 1.1800x over previous
import functools

import jax
import jax.numpy as jnp
from jax import lax
from jax.experimental import pallas as pl
from jax.experimental.pallas import tpu as pltpu
from jax.experimental.pallas import tpu_sc as plsc

_NCHARS = 256
_CE = 8
_WE = 16
_WLEN = 20
_OUT_D = _WE + _WLEN * _CE
_NTILES = 32
_K = 128


def _build_kernel(n_pairs: int):
    assert n_pairs % (_NTILES * 2 * _K) == 0
    p_per_tile = n_pairs // _NTILES
    n_blocks = p_per_tile // _K
    mesh = plsc.VectorSubcoreMesh(core_axis_name="c", subcore_axis_name="s")

    @functools.partial(
        pl.kernel,
        out_type=jax.ShapeDtypeStruct((n_pairs * _OUT_D,), jnp.float32),
        mesh=mesh,
        scratch_types=[
            pltpu.VMEM(((_NCHARS + 1) * _CE,), jnp.float32),
            pltpu.VMEM((_K * _WLEN,), jnp.int32),
            pltpu.VMEM((_K * _WLEN,), jnp.int32),
            pltpu.VMEM((_K,), jnp.int32),
            pltpu.VMEM((_K,), jnp.int32),
            pltpu.VMEM((_K, _WE), jnp.float32),
            pltpu.VMEM((_K, _WE), jnp.float32),
            pltpu.VMEM((_K * _OUT_D,), jnp.float32),
            pltpu.VMEM((_K * _OUT_D,), jnp.float32),
            pltpu.SemaphoreType.DMA,
            pltpu.SemaphoreType.DMA,
            pltpu.SemaphoreType.DMA,
            pltpu.SemaphoreType.DMA,
            pltpu.SemaphoreType.DMA,
            pltpu.SemaphoreType.DMA,
            pltpu.SemaphoreType.DMA,
        ],
        compiler_params=pltpu.CompilerParams(
            needs_layout_passes=False, use_tc_tiling_on_sc=False),
    )
    def _k(cid_hbm, wid_hbm, ctab_hbm, wtab_hbm, out_hbm,
           ctab_v, cid0_v, cid1_v, wid0_v, wid1_v, wrows0_v, wrows1_v,
           outb0_v, outb1_v, tsem, isem0, isem1, gsem0, gsem1, osem0, osem1):
        cid_b = (cid0_v, cid1_v)
        wid_b = (wid0_v, wid1_v)
        wrows_b = (wrows0_v, wrows1_v)
        outb_b = (outb0_v, outb1_v)
        isem_b = (isem0, isem1)
        gsem_b = (gsem0, gsem1)
        osem_b = (osem0, osem1)
        tid = lax.axis_index("s") * 2 + lax.axis_index("c")
        tbase = tid * p_per_tile

        iota = lax.iota(jnp.int32, 16)
        lo8 = iota & 7
        hi8 = iota >> 3
        zeros = jnp.zeros((16,), jnp.float32)

        pltpu.async_copy(ctab_hbm, ctab_v, tsem).wait()
        head = ctab_v[pl.ds(0, 16)]
        ctab_v[pl.ds(0, 16)] = jnp.where(iota < _CE, 0.0, head)

        def start_ids(b, h):
            pbase = tbase + b * _K
            pltpu.async_copy(
                cid_hbm.at[pl.ds(pbase * _WLEN, _K * _WLEN)], cid_b[h],
                isem_b[h])
            pltpu.async_copy(
                wid_hbm.at[pl.ds(pbase, _K)], wid_b[h], isem_b[h])

        def wait_ids(h):
            pltpu.make_async_copy(
                cid_hbm.at[pl.ds(0, _K * _WLEN)], cid_b[h], isem_b[h]).wait()
            pltpu.make_async_copy(
                wid_hbm.at[pl.ds(0, _K)], wid_b[h], isem_b[h]).wait()

        def start_gather(h):
            pltpu.async_copy(wtab_hbm.at[wid_b[h]], wrows_b[h], gsem_b[h])

        def wait_gather(h):
            pltpu.make_async_copy(wtab_hbm.at[wid_b[h]], wrows_b[h],
                                  gsem_b[h]).wait()

        def start_out(b, h):
            pbase = tbase + b * _K
            pltpu.async_copy(
                outb_b[h],
                out_hbm.at[pl.ds(pbase * _OUT_D, _K * _OUT_D)], osem_b[h])

        def wait_out(b, h):
            pbase = tbase + b * _K
            pltpu.make_async_copy(
                outb_b[h],
                out_hbm.at[pl.ds(pbase * _OUT_D, _K * _OUT_D)],
                osem_b[h]).wait()

        start_ids(0, 0)
        wait_ids(0)
        start_gather(0)
        start_ids(1, 1)

        def block2(b2, carry):
            for h in (0, 1):
                b = b2 * 2 + h
                wait_gather(h)
                @pl.when(b >= 2)
                def _():
                    wait_out(b - 2, h)

                cid_h = cid_b[h]
                wid_h = wid_b[h]
                wrows_h = wrows_b[h]
                outb_h = outb_b[h]

                def grp(g, c):
                    obase = (g * 16 + iota) * _OUT_D
                    for p16 in range(16):
                        p = g * 16 + p16
                        outb_h[pl.ds(p * _OUT_D, 16)] = wrows_h[p]
                    widv = wid_h[pl.ds(g * 16, 16)]
                    m = widv == 0
                    for col in range(_WE):
                        plsc.store_scatter(outb_h, [obase + col], zeros,
                                           mask=m)
                    cbase = (g * 16 + iota) * _WLEN
                    for t in range(_WLEN):
                        cidt = plsc.load_gather(cid_h, [cbase + t])
                        ebase = cidt * _CE
                        tout = obase + _WE + t * _CE
                        for d in range(_CE):
                            val = plsc.load_gather(ctab_v, [ebase + d])
                            plsc.store_scatter(outb_h, [tout + d], val)
                    return c
                lax.fori_loop(0, _K // 16, grp, 0)

                start_out(b, h)

                @pl.when(b + 1 < n_blocks)
                def _():
                    wait_ids(1 - h)
                    start_gather(1 - h)

                @pl.when(b + 2 < n_blocks)
                def _():
                    start_ids(b + 2, h)
            return carry

        lax.fori_loop(0, n_blocks // 2, block2, 0)

        wait_out(n_blocks - 2, 0)
        wait_out(n_blocks - 1, 1)

    return _k


@jax.jit
def kernel(char_ids, word_ids, char_table, word_table):
    b, s, w = char_ids.shape
    n_pairs = b * s
    k = _build_kernel(n_pairs)
    out = k(char_ids.reshape(-1), word_ids.reshape(-1),
            char_table.reshape(-1), word_table)
    return out.reshape(b, s, _OUT_D)


# speedup vs baseline: 29.0615x; 1.5104x over previous
import functools

import jax
import jax.numpy as jnp
from jax import lax
from jax.experimental import pallas as pl
from jax.experimental.pallas import tpu as pltpu
from jax.experimental.pallas import tpu_sc as plsc

_NCHARS = 256
_CE = 8
_WE = 16
_WLEN = 20
_OUT_D = _WE + _WLEN * _CE
_NTILES = 32
_K = 128


def _build_kernel(n_pairs: int):
    assert n_pairs % (_NTILES * 2 * _K) == 0
    p_per_tile = n_pairs // _NTILES
    n_blocks = p_per_tile // _K
    mesh = plsc.VectorSubcoreMesh(core_axis_name="c", subcore_axis_name="s")

    @functools.partial(
        pl.kernel,
        out_type=jax.ShapeDtypeStruct((n_pairs * _OUT_D,), jnp.float32),
        mesh=mesh,
        scratch_types=[
            pltpu.VMEM(((_NCHARS + 1) * _CE,), jnp.float32),
            pltpu.VMEM((_K * _WLEN,), jnp.int32),
            pltpu.VMEM((_K * _WLEN,), jnp.int32),
            pltpu.VMEM((_K,), jnp.int32),
            pltpu.VMEM((_K,), jnp.int32),
            pltpu.VMEM((_K, _WE), jnp.float32),
            pltpu.VMEM((_K, _WE), jnp.float32),
            pltpu.VMEM((_K * _OUT_D,), jnp.float32),
            pltpu.VMEM((_K * _OUT_D,), jnp.float32),
            pltpu.SemaphoreType.DMA,
            pltpu.SemaphoreType.DMA,
            pltpu.SemaphoreType.DMA,
            pltpu.SemaphoreType.DMA,
            pltpu.SemaphoreType.DMA,
            pltpu.SemaphoreType.DMA,
            pltpu.SemaphoreType.DMA,
        ],
        compiler_params=pltpu.CompilerParams(
            needs_layout_passes=False, use_tc_tiling_on_sc=False),
    )
    def _k(cid_hbm, wid_hbm, ctab_hbm, wtab_hbm, out_hbm,
           ctab_v, cid0_v, cid1_v, wid0_v, wid1_v, wrows0_v, wrows1_v,
           outb0_v, outb1_v, tsem, isem0, isem1, gsem0, gsem1, osem0, osem1):
        cid_b = (cid0_v, cid1_v)
        wid_b = (wid0_v, wid1_v)
        wrows_b = (wrows0_v, wrows1_v)
        outb_b = (outb0_v, outb1_v)
        isem_b = (isem0, isem1)
        gsem_b = (gsem0, gsem1)
        osem_b = (osem0, osem1)
        tid = lax.axis_index("s") * 2 + lax.axis_index("c")
        tbase = tid * p_per_tile

        iota = lax.iota(jnp.int32, 16)
        lo8 = iota & 7
        hi8 = iota >> 3
        zeros = jnp.zeros((16,), jnp.float32)

        pltpu.async_copy(ctab_hbm, ctab_v, tsem).wait()
        head = ctab_v[pl.ds(0, 16)]
        ctab_v[pl.ds(0, 16)] = jnp.where(iota < _CE, 0.0, head)

        def start_ids(b, h):
            pbase = tbase + b * _K
            pltpu.async_copy(
                cid_hbm.at[pl.ds(pbase * _WLEN, _K * _WLEN)], cid_b[h],
                isem_b[h])
            pltpu.async_copy(
                wid_hbm.at[pl.ds(pbase, _K)], wid_b[h], isem_b[h])

        def wait_ids(h):
            pltpu.make_async_copy(
                cid_hbm.at[pl.ds(0, _K * _WLEN)], cid_b[h], isem_b[h]).wait()
            pltpu.make_async_copy(
                wid_hbm.at[pl.ds(0, _K)], wid_b[h], isem_b[h]).wait()

        def start_gather(h):
            pltpu.async_copy(wtab_hbm.at[wid_b[h]], wrows_b[h], gsem_b[h])

        def wait_gather(h):
            pltpu.make_async_copy(wtab_hbm.at[wid_b[h]], wrows_b[h],
                                  gsem_b[h]).wait()

        def start_out(b, h):
            pbase = tbase + b * _K
            pltpu.async_copy(
                outb_b[h],
                out_hbm.at[pl.ds(pbase * _OUT_D, _K * _OUT_D)], osem_b[h])

        def wait_out(b, h):
            pbase = tbase + b * _K
            pltpu.make_async_copy(
                outb_b[h],
                out_hbm.at[pl.ds(pbase * _OUT_D, _K * _OUT_D)],
                osem_b[h]).wait()

        start_ids(0, 0)
        wait_ids(0)
        start_gather(0)
        start_ids(1, 1)

        def block2(b2, carry):
            for h in (0, 1):
                b = b2 * 2 + h
                wait_gather(h)
                @pl.when(b >= 2)
                def _():
                    wait_out(b - 2, h)

                cid_h = cid_b[h]
                wid_h = wid_b[h]
                wrows_h = wrows_b[h]
                outb_h = outb_b[h]

                @plsc.parallel_loop(0, _K // 16, 1, unroll=2)
                def grp(g):
                    obase = (g * 16 + iota) * _OUT_D
                    for p16 in range(16):
                        p = g * 16 + p16
                        outb_h[pl.ds(p * _OUT_D, 16)] = wrows_h[p]
                    widv = wid_h[pl.ds(g * 16, 16)]
                    m = widv == 0
                    for col in range(_WE):
                        plsc.store_scatter(outb_h, [obase + col], zeros,
                                           mask=m)
                    cbase = (g * 16 + iota) * _WLEN
                    cids = [plsc.load_gather(cid_h, [cbase + t])
                            for t in range(_WLEN)]
                    for t in range(_WLEN):
                        ebase = cids[t] * _CE
                        tout = obase + _WE + t * _CE
                        vals = [plsc.load_gather(ctab_v, [ebase + d])
                                for d in range(_CE)]
                        for d in range(_CE):
                            plsc.store_scatter(outb_h, [tout + d], vals[d])

                start_out(b, h)

                @pl.when(b + 1 < n_blocks)
                def _():
                    wait_ids(1 - h)
                    start_gather(1 - h)

                @pl.when(b + 2 < n_blocks)
                def _():
                    start_ids(b + 2, h)
            return carry

        lax.fori_loop(0, n_blocks // 2, block2, 0)

        wait_out(n_blocks - 2, 0)
        wait_out(n_blocks - 1, 1)

    return _k


@jax.jit
def kernel(char_ids, word_ids, char_table, word_table):
    b, s, w = char_ids.shape
    n_pairs = b * s
    k = _build_kernel(n_pairs)
    out = k(char_ids.reshape(-1), word_ids.reshape(-1),
            char_table.reshape(-1), word_table)
    return out.reshape(b, s, _OUT_D)


# speedup vs baseline: 31.9545x; 1.0995x over previous
import functools

import jax
import jax.numpy as jnp
from jax import lax
from jax.experimental import pallas as pl
from jax.experimental.pallas import tpu as pltpu
from jax.experimental.pallas import tpu_sc as plsc

_NCHARS = 256
_CE = 8
_WE = 16
_WLEN = 20
_OUT_D = _WE + _WLEN * _CE
_NTILES = 32
_K = 128

_DNUMS = lax.GatherDimensionNumbers(
    offset_dims=(), collapsed_slice_dims=(0,), start_index_map=(0,))


def _build_kernel(n_pairs: int):
    assert n_pairs % (_NTILES * 2 * _K) == 0
    p_per_tile = n_pairs // _NTILES
    n_blocks = p_per_tile // _K
    mesh = plsc.VectorSubcoreMesh(core_axis_name="c", subcore_axis_name="s")

    @functools.partial(
        pl.kernel,
        out_type=jax.ShapeDtypeStruct((n_pairs * _OUT_D,), jnp.float32),
        mesh=mesh,
        scratch_types=[
            pltpu.VMEM(((_NCHARS + 1) * _CE,), jnp.float32),
            pltpu.VMEM((_K * _WLEN,), jnp.int32),
            pltpu.VMEM((_K * _WLEN,), jnp.int32),
            pltpu.VMEM((_K,), jnp.int32),
            pltpu.VMEM((_K,), jnp.int32),
            pltpu.VMEM((_K, _WE), jnp.float32),
            pltpu.VMEM((_K, _WE), jnp.float32),
            pltpu.VMEM((_K * _OUT_D,), jnp.float32),
            pltpu.VMEM((_K * _OUT_D,), jnp.float32),
            pltpu.SemaphoreType.DMA,
            pltpu.SemaphoreType.DMA,
            pltpu.SemaphoreType.DMA,
            pltpu.SemaphoreType.DMA,
            pltpu.SemaphoreType.DMA,
            pltpu.SemaphoreType.DMA,
            pltpu.SemaphoreType.DMA,
        ],
        compiler_params=pltpu.CompilerParams(
            needs_layout_passes=False, use_tc_tiling_on_sc=False),
    )
    def _k(cid_hbm, wid_hbm, ctab_hbm, wtab_hbm, out_hbm,
           ctab_v, cid0_v, cid1_v, wid0_v, wid1_v, wrows0_v, wrows1_v,
           outb0_v, outb1_v, tsem, isem0, isem1, gsem0, gsem1, osem0, osem1):
        cid_b = (cid0_v, cid1_v)
        wid_b = (wid0_v, wid1_v)
        wrows_b = (wrows0_v, wrows1_v)
        outb_b = (outb0_v, outb1_v)
        isem_b = (isem0, isem1)
        gsem_b = (gsem0, gsem1)
        osem_b = (osem0, osem1)
        tid = lax.axis_index("s") * 2 + lax.axis_index("c")
        tbase = tid * p_per_tile

        iota = lax.iota(jnp.int32, 16)
        lo8 = iota & 7
        hi8 = iota >> 3
        zeros = jnp.zeros((16,), jnp.float32)

        pltpu.async_copy(ctab_hbm, ctab_v, tsem).wait()
        head = ctab_v[pl.ds(0, 16)]
        ctab_v[pl.ds(0, 16)] = jnp.where(iota < _CE, 0.0, head)

        def start_ids(b, h):
            pbase = tbase + b * _K
            pltpu.async_copy(
                cid_hbm.at[pl.ds(pbase * _WLEN, _K * _WLEN)], cid_b[h],
                isem_b[h])
            pltpu.async_copy(
                wid_hbm.at[pl.ds(pbase, _K)], wid_b[h], isem_b[h])

        def wait_ids(h):
            pltpu.make_async_copy(
                cid_hbm.at[pl.ds(0, _K * _WLEN)], cid_b[h], isem_b[h]).wait()
            pltpu.make_async_copy(
                wid_hbm.at[pl.ds(0, _K)], wid_b[h], isem_b[h]).wait()

        def start_gather(h):
            pltpu.async_copy(wtab_hbm.at[wid_b[h]], wrows_b[h], gsem_b[h])

        def wait_gather(h):
            pltpu.make_async_copy(wtab_hbm.at[wid_b[h]], wrows_b[h],
                                  gsem_b[h]).wait()

        def start_out(b, h):
            pbase = tbase + b * _K
            pltpu.async_copy(
                outb_b[h],
                out_hbm.at[pl.ds(pbase * _OUT_D, _K * _OUT_D)], osem_b[h])

        def wait_out(b, h):
            pbase = tbase + b * _K
            pltpu.make_async_copy(
                outb_b[h],
                out_hbm.at[pl.ds(pbase * _OUT_D, _K * _OUT_D)],
                osem_b[h]).wait()

        start_ids(0, 0)
        wait_ids(0)
        start_gather(0)
        start_ids(1, 1)

        def block2(b2, carry):
            for h in (0, 1):
                b = b2 * 2 + h
                wait_gather(h)
                @pl.when(b >= 2)
                def _():
                    wait_out(b - 2, h)

                cid_h = cid_b[h]
                wid_h = wid_b[h]
                wrows_h = wrows_b[h]
                outb_h = outb_b[h]

                @plsc.parallel_loop(0, _K, 1, unroll=2)
                def pairloop(p):
                    ob = p * _OUT_D
                    cb = p * _WLEN
                    outb_h[pl.ds(ob, 16)] = wrows_h[p]
                    v0 = cid_h[pl.ds(cb, 16)]
                    v1 = cid_h[pl.ds(cb + 4, 16)]
                    vals = []
                    for j in range(_WLEN // 2):
                        if j < 8:
                            src, base = v0, 2 * j
                        else:
                            src, base = v1, 12 + 2 * (j - 8)
                        pat = base + hi8
                        cidj = lax.gather(
                            src, pat[:, None], _DNUMS, (1,),
                            mode=lax.GatherScatterMode.PROMISE_IN_BOUNDS)
                        vals.append(
                            plsc.load_gather(ctab_v, [cidj * _CE + lo8]))
                    for j in range(_WLEN // 2):
                        outb_h[pl.ds(ob + _WE + 16 * j, 16)] = vals[j]

                @plsc.parallel_loop(0, _K // 16, 1)
                def zgrp(g):
                    obase = (g * 16 + iota) * _OUT_D
                    widv = wid_h[pl.ds(g * 16, 16)]
                    m = widv == 0
                    for col in range(_WE):
                        plsc.store_scatter(outb_h, [obase + col], zeros,
                                           mask=m)

                start_out(b, h)

                @pl.when(b + 1 < n_blocks)
                def _():
                    wait_ids(1 - h)
                    start_gather(1 - h)

                @pl.when(b + 2 < n_blocks)
                def _():
                    start_ids(b + 2, h)
            return carry

        lax.fori_loop(0, n_blocks // 2, block2, 0)

        wait_out(n_blocks - 2, 0)
        wait_out(n_blocks - 1, 1)

    return _k


@jax.jit
def kernel(char_ids, word_ids, char_table, word_table):
    b, s, w = char_ids.shape
    n_pairs = b * s
    k = _build_kernel(n_pairs)
    out = k(char_ids.reshape(-1), word_ids.reshape(-1),
            char_table.reshape(-1), word_table)
    return out.reshape(b, s, _OUT_D)
